# Initial kernel scaffold; baseline (speedup 1.0000x reference)
import jax
import jax.numpy as jnp
from jax.experimental import pallas as pl


def kernel(z, centers):
    raise NotImplementedError("write your pallas kernel here")


# speedup vs baseline: 1.4139x; 1.4139x over previous
import functools

import jax
import jax.numpy as jnp
from jax import lax
from jax.experimental import pallas as pl
from jax.experimental.pallas import tpu as pltpu

N, D, K = 18432, 64, 1024
BM = 512


def _assign_body(z_ref, c_ref, codes_ref, q_ref):
    z = z_ref[...]
    c = c_ref[...]
    d = lax.dot_general(z, c, (((1,), (1,)), ((), ())),
                        preferred_element_type=jnp.float32)
    d = d * (-2.0)
    d = d + jnp.sum(z * z, axis=1, keepdims=True)
    d = d + jnp.sum(c * c, axis=1)[None, :]
    codes = jnp.argmin(d, axis=1).astype(jnp.int32)
    codes_ref[0, 0, :] = codes
    onehot = (codes[:, None] == lax.broadcasted_iota(jnp.int32, (BM, K), 1))
    q_ref[...] = lax.dot_general(onehot.astype(jnp.float32), c,
                                 (((1,), (0,)), ((), ())),
                                 preferred_element_type=jnp.float32)


@jax.jit
def kernel(z, centers):
    grid = N // BM
    codes3, quant = pl.pallas_call(
        _assign_body,
        grid=(grid,),
        in_specs=[
            pl.BlockSpec((BM, D), lambda i: (i, 0)),
            pl.BlockSpec((K, D), lambda i: (0, 0)),
        ],
        out_specs=[
            pl.BlockSpec((1, 1, BM), lambda i: (i, 0, 0)),
            pl.BlockSpec((BM, D), lambda i: (i, 0)),
        ],
        out_shape=[
            jax.ShapeDtypeStruct((grid, 1, BM), jnp.int32),
            jax.ShapeDtypeStruct((N, D), jnp.float32),
        ],
    )(z, centers)
    return codes3.reshape(N), quant
